# baseline (speedup 1.0000x reference)
import jax
import jax.numpy as jnp
from jax.experimental import pallas as pl


def _gc(feats, src, dst, W, n):
    msg = jnp.take(feats, src, axis=0)
    agg = jax.ops.segment_sum(msg, dst, num_segments=n)
    deg = jax.ops.segment_sum(jnp.ones((src.shape[0], 1), feats.dtype), dst, num_segments=n)
    h = jnp.dot(feats + agg / jnp.maximum(deg, 1.0), W)
    return jax.nn.relu(h)


def _pl(node, feats, src, dst, pool_size, max_dim, self_loop):
    G = max_dim // pool_size
    c = node // pool_size
    vid = c[:, 0] * G * G + c[:, 1] * G + c[:, 2]
    nseg = G * G * G
    pooled = jax.ops.segment_max(feats, vid, num_segments=nseg)
    pooled = jnp.where(jnp.isfinite(pooled), pooled, 0.0)
    gx, gy, gz = jnp.meshgrid(jnp.arange(G), jnp.arange(G), jnp.arange(G), indexing='ij')
    new_node = jnp.stack([gx.ravel(), gy.ravel(), gz.ravel()], axis=1)
    new_src = jnp.take(vid, src)
    new_dst = jnp.take(vid, dst)
    if self_loop:
        loop = jnp.arange(nseg)
        new_src = jnp.concatenate([new_src, loop])
        new_dst = jnp.concatenate([new_dst, loop])
    return new_node, pooled, new_src, new_dst, nseg


def _linear_body(x_ref, w_ref, b_ref, o_ref):
    o_ref[...] = jnp.dot(x_ref[...], w_ref[...],
                         preferred_element_type=jnp.float32) + b_ref[...]


def _linear(flat, Wlin, blin):
    return pl.pallas_call(
        _linear_body,
        out_shape=jax.ShapeDtypeStruct((1, 100), jnp.float32),
    )(flat.reshape(1, -1), Wlin, blin.reshape(1, -1)).reshape(-1)


def kernel(node, features, edges, W1, W2, W3, W4, W5, W6, W7, Wlin, blin):
    src, dst = edges[0], edges[1]
    n = node.shape[0]
    h = _gc(features, src, dst, W1, n)
    h = _gc(h, src, dst, W2, n)
    node, h, src, dst, n = _pl(node, h, src, dst, 4, 256, True)
    h = _gc(h, src, dst, W3, n)
    h = _gc(h, src, dst, W4, n)
    h = _gc(h, src, dst, W5, n)
    node, h, src, dst, n = _pl(node, h, src, dst, 4, 64, False)
    h = _gc(h, src, dst, W6, n)
    h = _gc(h, src, dst, W7, n)
    G = 16 // 4
    c = node // 4
    vid = c[:, 0] * G * G + c[:, 1] * G + c[:, 2]
    pooled = jax.ops.segment_max(h, vid, num_segments=G * G * G)
    pooled = jnp.where(jnp.isfinite(pooled), pooled, 0.0)
    flat = pooled.reshape(-1)
    return _linear(flat, Wlin, blin)


# speedup vs baseline: 1.3785x; 1.3785x over previous
import numpy as np
import jax
import jax.numpy as jnp
from jax.experimental import pallas as pl

_NS = 50000


def _seg_sum(vals, seg, n):
    return jax.ops.segment_sum(vals, seg, num_segments=n)


def _seg_max0(vals, seg, n):
    m = jax.ops.segment_max(vals, seg, num_segments=n)
    return jnp.where(jnp.isfinite(m), m, 0.0)


def _gconv_xla(h, src, dst, W, n, deg):
    agg = _seg_sum(jnp.take(h, src, axis=0), dst, n)
    z = h + agg / jnp.maximum(deg, 1.0)
    return jax.nn.relu(jnp.dot(z, W, preferred_element_type=jnp.float32))


def _linear_body(x_ref, w_ref, b_ref, o_ref):
    o_ref[...] = jnp.dot(x_ref[...], w_ref[...],
                         preferred_element_type=jnp.float32) + b_ref[...]


def _linear(flat, Wlin, blin):
    return pl.pallas_call(
        _linear_body,
        out_shape=jax.ShapeDtypeStruct((1, 100), jnp.float32),
    )(flat.reshape(1, -1), Wlin, blin.reshape(1, -1)).reshape(-1)


def _final_perm():
    r = np.arange(4096)
    x, y, z = r // 256, (r // 16) % 16, r % 16
    j = (x // 4) * 16 + (y // 4) * 4 + (z // 4)
    i = (x % 4) * 16 + (y % 4) * 4 + (z % 4)
    perm = np.zeros(4096, dtype=np.int32)
    perm[j * 64 + i] = r
    return jnp.asarray(perm)


def kernel(node, features, edges, W1, W2, W3, W4, W5, W6, W7, Wlin, blin):
    src, dst = edges[0], edges[1]
    n = node.shape[0]

    deg_a = _seg_sum(jnp.ones((src.shape[0], 1), jnp.float32), dst, n)
    h1 = _gconv_xla(features, src, dst, W1, n, deg_a)
    h2 = _gconv_xla(h1, src, dst, W2, n, deg_a)

    c = node // 4
    vid = c[:, 0] * 4096 + c[:, 1] * 64 + c[:, 2]
    occ = jnp.zeros((262144,), jnp.int32).at[vid].set(1)
    cidx_of_voxel = jnp.cumsum(occ) - occ
    cid = jnp.take(cidx_of_voxel, vid)
    pooled1 = _seg_max0(h2, cid, _NS)

    cc = node // 16
    vid2 = cc[:, 0] * 256 + cc[:, 1] * 16 + cc[:, 2]
    slot_vid2 = jnp.full((_NS,), 4096, jnp.int32).at[cid].set(vid2)

    csrc = jnp.take(cid, src)
    cdst = jnp.take(cid, dst)
    loop = jnp.arange(_NS, dtype=csrc.dtype)
    bsrc = jnp.concatenate([csrc, loop])
    bdst = jnp.concatenate([cdst, loop])

    deg_b = _seg_sum(jnp.ones((bsrc.shape[0], 1), jnp.float32), bdst, _NS)
    h3 = _gconv_xla(pooled1, bsrc, bdst, W3, _NS, deg_b)
    h4 = _gconv_xla(h3, bsrc, bdst, W4, _NS, deg_b)
    h5 = _gconv_xla(h4, bsrc, bdst, W5, _NS, deg_b)

    pooled2 = _seg_max0(h5, slot_vid2, 4097)[:4096]

    vsrc2 = jnp.take(vid2, src)
    vdst2 = jnp.take(vid2, dst)
    flat_id = vdst2 * 4096 + vsrc2
    C2 = _seg_sum(jnp.ones_like(flat_id, dtype=jnp.float32), flat_id,
                  4096 * 4096).reshape(4096, 4096)
    C2 = C2 + 64.0 * jnp.eye(4096, dtype=jnp.float32)
    deg6 = jnp.maximum(C2.sum(axis=1, keepdims=True), 1.0)
    h6 = jax.nn.relu(jnp.dot(pooled2 + jnp.dot(C2, pooled2) / deg6, W6,
                             preferred_element_type=jnp.float32))
    h7 = jax.nn.relu(jnp.dot(h6 + jnp.dot(C2, h6) / deg6, W7,
                             preferred_element_type=jnp.float32))

    hp = jnp.take(h7, _final_perm(), axis=0).reshape(64, 64, 64)
    pooled3 = jnp.max(hp, axis=1)
    return _linear(pooled3.reshape(-1), Wlin, blin)


# speedup vs baseline: 2.6143x; 1.8965x over previous
import functools
import numpy as np
import jax
import jax.numpy as jnp
from jax import lax
from jax.experimental import pallas as pl
from jax.experimental.pallas import tpu as pltpu
from jax.experimental.pallas import tpu_sc as plsc

_NS = 50000
_SPAD = 50048
_DUMMY = 50047
_BLK = 128
_ZCH = 136


def _sc_segsum(table, srci, dsti, F):
    E = srci.shape[0]
    ept = E // 32
    nb = ept // _BLK
    rows_pt = _SPAD // 16
    nz = rows_pt // _ZCH
    mesh = plsc.VectorSubcoreMesh(core_axis_name="c", subcore_axis_name="s")

    @functools.partial(
        pl.kernel, mesh=mesh,
        compiler_params=pltpu.CompilerParams(use_tc_tiling_on_sc=False),
        out_type=jax.ShapeDtypeStruct((2 * _SPAD, F), jnp.float32),
        scratch_types=[
            pltpu.VMEM((_BLK,), jnp.int32),
            pltpu.VMEM((_BLK,), jnp.int32),
            pltpu.VMEM((_BLK, F), jnp.float32),
            pltpu.VMEM((_ZCH, F), jnp.float32),
            pltpu.VMEM_SHARED((_SPAD, F), jnp.float32),
            pltpu.SemaphoreType.DMA,
        ],
    )
    def k(t_hbm, src_hbm, dst_hbm, out_hbm, idxs_v, idxd_v, rows_v, zer_v,
          acc_sh, sem):
        cid = lax.axis_index("c")
        sid = lax.axis_index("s")

        def zfill(i, _):
            r = i // (F // 16)
            c = (i % (F // 16)) * 16
            zer_v[r, pl.ds(c, 16)] = jnp.zeros((16,), jnp.float32)
            return 0
        lax.fori_loop(0, _ZCH * (F // 16), zfill, 0)

        row0 = sid * rows_pt

        def zcopy(i, _):
            pltpu.sync_copy(zer_v, acc_sh.at[pl.ds(row0 + i * _ZCH, _ZCH)])
            return 0
        lax.fori_loop(0, nz, zcopy, 0)
        plsc.subcore_barrier()

        base = (cid * 16 + sid) * ept

        def body(b, _):
            off = base + b * _BLK
            pltpu.sync_copy(src_hbm.at[pl.ds(off, _BLK)], idxs_v)
            pltpu.sync_copy(dst_hbm.at[pl.ds(off, _BLK)], idxd_v)
            pltpu.async_copy(t_hbm.at[idxs_v], rows_v, sem).wait()
            pltpu.sync_copy(rows_v, acc_sh.at[idxd_v], add=True)
            return 0
        lax.fori_loop(0, nb, body, 0)
        plsc.subcore_barrier()

        obase = cid * _SPAD + row0

        def wcopy(i, _):
            pltpu.sync_copy(acc_sh.at[pl.ds(row0 + i * _ZCH, _ZCH)],
                            out_hbm.at[pl.ds(obase + i * _ZCH, _ZCH)])
            return 0
        lax.fori_loop(0, nz, wcopy, 0)

    return k(table, srci, dsti)


def _segsum(table, srci, dsti, F):
    p = _sc_segsum(table, srci, dsti, F).reshape(2, _SPAD, F)
    return (p[0] + p[1])[:_NS]


def _pad_edges(s, d, e_pad):
    npad = e_pad - s.shape[0]
    return (jnp.concatenate([s, jnp.zeros((npad,), s.dtype)]),
            jnp.concatenate([d, jnp.full((npad,), _DUMMY, d.dtype)]))


def _seg_max0(vals, seg, n):
    m = jax.ops.segment_max(vals, seg, num_segments=n)
    return jnp.where(jnp.isfinite(m), m, 0.0)


def _linear_body(x_ref, w_ref, b_ref, o_ref):
    o_ref[...] = jnp.dot(x_ref[...], w_ref[...],
                         preferred_element_type=jnp.float32) + b_ref[...]


def _linear(flat, Wlin, blin):
    return pl.pallas_call(
        _linear_body,
        out_shape=jax.ShapeDtypeStruct((1, 100), jnp.float32),
    )(flat.reshape(1, -1), Wlin, blin.reshape(1, -1)).reshape(-1)


def _final_perm():
    r = np.arange(4096)
    x, y, z = r // 256, (r // 16) % 16, r % 16
    j = (x // 4) * 16 + (y // 4) * 4 + (z // 4)
    i = (x % 4) * 16 + (y % 4) * 4 + (z % 4)
    perm = np.zeros(4096, dtype=np.int32)
    perm[j * 64 + i] = r
    return jnp.asarray(perm)


def kernel(node, features, edges, W1, W2, W3, W4, W5, W6, W7, Wlin, blin):
    src, dst = edges[0], edges[1]
    nn = node.shape[0]
    f32 = jnp.float32

    srcA, dstA = _pad_edges(src, dst, 32 * _BLK * 200)
    zeros14 = jnp.zeros((nn, 14), f32)
    ones1 = jnp.ones((nn, 1), f32)
    tA = jnp.concatenate([features, ones1, zeros14], axis=1)
    sA = _segsum(tA, srcA, dstA, 16)
    deg_a = jnp.maximum(sA[:, 1:2], 1.0)
    z = features + sA[:, 0:1] / deg_a
    u = jnp.concatenate([jax.nn.relu(z), jax.nn.relu(-z), ones1, zeros14[:, :13]],
                        axis=1)
    sU = _segsum(u, srcA, dstA, 16)
    u2 = u[:, 0:2]
    m2 = jnp.concatenate([jnp.maximum(W1, 0.0), -jnp.minimum(W1, 0.0)], axis=0)
    h2 = jax.nn.relu(jnp.dot(u2 + sU[:, 0:2] / deg_a, jnp.dot(m2, W2),
                             preferred_element_type=f32))

    c = node // 4
    vid = c[:, 0] * 4096 + c[:, 1] * 64 + c[:, 2]
    occ = jnp.zeros((262144,), jnp.int32).at[vid].set(1)
    cidx_of_voxel = jnp.cumsum(occ) - occ
    cid = jnp.take(cidx_of_voxel, vid)
    pooled1 = _seg_max0(h2, cid, _NS)

    cc = node // 16
    vid2 = cc[:, 0] * 256 + cc[:, 1] * 16 + cc[:, 2]
    slot_vid2 = jnp.full((_NS,), 4096, jnp.int32).at[cid].set(vid2)

    loop = jnp.arange(_NS, dtype=src.dtype)
    bsrc = jnp.concatenate([jnp.take(cid, src), loop])
    bdst = jnp.concatenate([jnp.take(cid, dst), loop])
    srcB, dstB = _pad_edges(bsrc, bdst, 32 * _BLK * 208)

    tB = jnp.concatenate([pooled1, jnp.ones((_NS, 1), f32),
                          jnp.zeros((_NS, 15), f32)], axis=1)
    s3 = _segsum(tB, srcB, dstB, 32)
    deg_b = jnp.maximum(s3[:, 16:17], 1.0)
    h3 = jax.nn.relu(jnp.dot(pooled1 + s3[:, :16] / deg_b, W3,
                             preferred_element_type=f32))
    s4 = _segsum(h3, srcB, dstB, 32)
    h4 = jax.nn.relu(jnp.dot(h3 + s4 / deg_b, W4, preferred_element_type=f32))
    s5 = _segsum(h4, srcB, dstB, 32)
    h5 = jax.nn.relu(jnp.dot(h4 + s5 / deg_b, W5, preferred_element_type=f32))

    pooled2 = _seg_max0(h5, slot_vid2, 4097)[:4096]

    vsrc2 = jnp.take(vid2, src)
    vdst2 = jnp.take(vid2, dst)
    flat_id = vdst2 * 4096 + vsrc2
    C2 = jax.ops.segment_sum(jnp.ones_like(flat_id, dtype=f32), flat_id,
                             num_segments=4096 * 4096).reshape(4096, 4096)
    C2 = C2 + 64.0 * jnp.eye(4096, dtype=f32)
    deg6 = jnp.maximum(C2.sum(axis=1, keepdims=True), 1.0)
    h6 = jax.nn.relu(jnp.dot(pooled2 + jnp.dot(C2, pooled2) / deg6, W6,
                             preferred_element_type=f32))
    h7 = jax.nn.relu(jnp.dot(h6 + jnp.dot(C2, h6) / deg6, W7,
                             preferred_element_type=f32))

    hp = jnp.take(h7, _final_perm(), axis=0).reshape(64, 64, 64)
    pooled3 = jnp.max(hp, axis=1)
    return _linear(pooled3.reshape(-1), Wlin, blin)


# speedup vs baseline: 10.6938x; 4.0905x over previous
import functools
import numpy as np
import jax
import jax.numpy as jnp
from jax import lax
from jax.experimental import pallas as pl
from jax.experimental.pallas import tpu as pltpu
from jax.experimental.pallas import tpu_sc as plsc

_NS = 50000
_SPAD = 50048
_DUMMY = 50047
_BLK = 128
_ZCH = 136


def _sc_segsum(table, srci, dsti, F, luts=None):
    E = srci.shape[0]
    ept = E // 32
    nb = ept // _BLK
    rows_pt = _SPAD // 16
    nz = rows_pt // _ZCH
    mesh = plsc.VectorSubcoreMesh(core_axis_name="c", subcore_axis_name="s")

    agg_ty = jax.ShapeDtypeStruct((2 * _SPAD, F), jnp.float32)
    e_i32 = jax.ShapeDtypeStruct((E,), jnp.int32)
    out_ty = agg_ty if luts is None else (agg_ty, e_i32, e_i32, e_i32)
    remap_scratch = [] if luts is None else [
        pltpu.VMEM((_BLK,), jnp.int32),
        pltpu.VMEM((_BLK,), jnp.int32),
        pltpu.VMEM((_BLK,), jnp.int32),
    ]

    @functools.partial(
        pl.kernel, mesh=mesh,
        compiler_params=pltpu.CompilerParams(use_tc_tiling_on_sc=False),
        out_type=out_ty,
        scratch_types=[
            pltpu.VMEM((_BLK,), jnp.int32),
            pltpu.VMEM((_BLK,), jnp.int32),
            pltpu.VMEM((_BLK, F), jnp.float32),
            pltpu.VMEM((_ZCH, F), jnp.float32),
            pltpu.VMEM_SHARED((_SPAD, F), jnp.float32),
            pltpu.SemaphoreType.DMA,
        ] + remap_scratch,
    )
    def k(t_hbm, src_hbm, dst_hbm, *rest):
        if luts is None:
            out_hbm, idxs_v, idxd_v, rows_v, zer_v, acc_sh, sem = rest
        else:
            (cid_hbm, vid2_hbm, out_hbm, cs_hbm, cd_hbm, fl_hbm,
             idxs_v, idxd_v, rows_v, zer_v, acc_sh, sem,
             cs_v, cd_v, fl_v) = rest
        cid = lax.axis_index("c")
        sid = lax.axis_index("s")

        def zfill(i, _):
            r = i // (F // 16)
            c = (i % (F // 16)) * 16
            zer_v[r, pl.ds(c, 16)] = jnp.zeros((16,), jnp.float32)
            return 0
        lax.fori_loop(0, _ZCH * (F // 16), zfill, 0)

        row0 = sid * rows_pt

        def zcopy(i, _):
            pltpu.sync_copy(zer_v, acc_sh.at[pl.ds(row0 + i * _ZCH, _ZCH)])
            return 0
        lax.fori_loop(0, nz, zcopy, 0)
        plsc.subcore_barrier()

        base = (cid * 16 + sid) * ept

        def body(b, _):
            off = base + b * _BLK
            pltpu.sync_copy(src_hbm.at[pl.ds(off, _BLK)], idxs_v)
            pltpu.sync_copy(dst_hbm.at[pl.ds(off, _BLK)], idxd_v)
            pltpu.async_copy(t_hbm.at[idxs_v], rows_v, sem).wait()
            pltpu.sync_copy(rows_v, acc_sh.at[idxd_v], add=True)
            if luts is not None:
                pltpu.async_copy(cid_hbm.at[idxs_v], cs_v, sem).wait()
                pltpu.async_copy(cid_hbm.at[idxd_v], cd_v, sem).wait()
                pltpu.sync_copy(cs_v, cs_hbm.at[pl.ds(off, _BLK)])
                pltpu.sync_copy(cd_v, cd_hbm.at[pl.ds(off, _BLK)])
                pltpu.async_copy(vid2_hbm.at[idxs_v], cs_v, sem).wait()
                pltpu.async_copy(vid2_hbm.at[idxd_v], cd_v, sem).wait()
                for j in range(_BLK // 16):
                    s16 = pl.ds(j * 16, 16)
                    fl_v[s16] = cd_v[s16] * 4096 + cs_v[s16]
                pltpu.sync_copy(fl_v, fl_hbm.at[pl.ds(off, _BLK)])
            return 0
        lax.fori_loop(0, nb, body, 0)
        plsc.subcore_barrier()

        obase = cid * _SPAD + row0

        def wcopy(i, _):
            pltpu.sync_copy(acc_sh.at[pl.ds(row0 + i * _ZCH, _ZCH)],
                            out_hbm.at[pl.ds(obase + i * _ZCH, _ZCH)])
            return 0
        lax.fori_loop(0, nz, wcopy, 0)

    if luts is None:
        return k(table, srci, dsti)
    return k(table, srci, dsti, luts[0], luts[1])


def _segsum(table, srci, dsti, F, luts=None):
    r = _sc_segsum(table, srci, dsti, F, luts)
    p = (r if luts is None else r[0]).reshape(2, _SPAD, F)
    agg = (p[0] + p[1])[:_NS]
    return agg if luts is None else (agg, r[1], r[2], r[3])


def _pad_edges(s, d, e_pad):
    npad = e_pad - s.shape[0]
    return (jnp.concatenate([s, jnp.zeros((npad,), s.dtype)]),
            jnp.concatenate([d, jnp.full((npad,), _DUMMY, d.dtype)]))


def _seg_max0(vals, seg, n):
    m = jax.ops.segment_max(vals, seg, num_segments=n)
    return jnp.where(jnp.isfinite(m), m, 0.0)


def _linear_body(x_ref, w_ref, b_ref, o_ref):
    o_ref[...] = jnp.dot(x_ref[...], w_ref[...],
                         preferred_element_type=jnp.float32) + b_ref[...]


def _linear(flat, Wlin, blin):
    return pl.pallas_call(
        _linear_body,
        out_shape=jax.ShapeDtypeStruct((1, 100), jnp.float32),
    )(flat.reshape(1, -1), Wlin, blin.reshape(1, -1)).reshape(-1)


def _final_perm():
    r = np.arange(4096)
    x, y, z = r // 256, (r // 16) % 16, r % 16
    j = (x // 4) * 16 + (y // 4) * 4 + (z // 4)
    i = (x % 4) * 16 + (y % 4) * 4 + (z % 4)
    perm = np.zeros(4096, dtype=np.int32)
    perm[j * 64 + i] = r
    return jnp.asarray(perm)


def kernel(node, features, edges, W1, W2, W3, W4, W5, W6, W7, Wlin, blin):
    src, dst = edges[0], edges[1]
    nn = node.shape[0]
    f32 = jnp.float32

    c = node // 4
    vid = c[:, 0] * 4096 + c[:, 1] * 64 + c[:, 2]
    occ = jnp.zeros((262144,), jnp.int32).at[vid].set(1)
    cidx_of_voxel = jnp.cumsum(occ) - occ
    cid_lut = jnp.take(cidx_of_voxel, vid).astype(jnp.int32)
    cc = node // 16
    vid2_lut = (cc[:, 0] * 256 + cc[:, 1] * 16 + cc[:, 2]).astype(jnp.int32)
    zpad = jnp.zeros((_SPAD - nn,), jnp.int32)
    luts = (jnp.concatenate([cid_lut, zpad]), jnp.concatenate([vid2_lut, zpad]))

    srcA, dstA = _pad_edges(src, dst, 32 * _BLK * 200)
    zeros14 = jnp.zeros((nn, 14), f32)
    ones1 = jnp.ones((nn, 1), f32)
    tA = jnp.concatenate([features, ones1, zeros14], axis=1)
    sA, csrcA, cdstA, flatA = _segsum(tA, srcA, dstA, 16, luts=luts)
    deg_a = jnp.maximum(sA[:, 1:2], 1.0)
    z = features + sA[:, 0:1] / deg_a
    u = jnp.concatenate([jax.nn.relu(z), jax.nn.relu(-z), ones1, zeros14[:, :13]],
                        axis=1)
    sU = _segsum(u, srcA, dstA, 16)
    u2 = u[:, 0:2]
    m2 = jnp.concatenate([jnp.maximum(W1, 0.0), -jnp.minimum(W1, 0.0)], axis=0)
    h2 = jax.nn.relu(jnp.dot(u2 + sU[:, 0:2] / deg_a, jnp.dot(m2, W2),
                             preferred_element_type=f32))

    pooled1 = _seg_max0(h2, cid_lut, _NS)
    slot_vid2 = jnp.full((_NS,), 4096, jnp.int32).at[cid_lut].set(vid2_lut)

    loop = jnp.arange(_NS, dtype=src.dtype)
    bsrc = jnp.concatenate([csrcA[:src.shape[0]], loop])
    bdst = jnp.concatenate([cdstA[:src.shape[0]], loop])
    srcB, dstB = _pad_edges(bsrc, bdst, 32 * _BLK * 208)

    tB = jnp.concatenate([pooled1, jnp.ones((_NS, 1), f32),
                          jnp.zeros((_NS, 15), f32)], axis=1)
    s3 = _segsum(tB, srcB, dstB, 32)
    deg_b = jnp.maximum(s3[:, 16:17], 1.0)
    h3 = jax.nn.relu(jnp.dot(pooled1 + s3[:, :16] / deg_b, W3,
                             preferred_element_type=f32))
    s4 = _segsum(h3, srcB, dstB, 32)
    h4 = jax.nn.relu(jnp.dot(h3 + s4 / deg_b, W4, preferred_element_type=f32))
    s5 = _segsum(h4, srcB, dstB, 32)
    h5 = jax.nn.relu(jnp.dot(h4 + s5 / deg_b, W5, preferred_element_type=f32))

    pooled2 = _seg_max0(h5, slot_vid2, 4097)[:4096]

    flat_id = flatA[:src.shape[0]]
    C2 = jax.ops.segment_sum(jnp.ones_like(flat_id, dtype=f32), flat_id,
                             num_segments=4096 * 4096).reshape(4096, 4096)
    C2 = C2 + 64.0 * jnp.eye(4096, dtype=f32)
    deg6 = jnp.maximum(C2.sum(axis=1, keepdims=True), 1.0)
    h6 = jax.nn.relu(jnp.dot(pooled2 + jnp.dot(C2, pooled2) / deg6, W6,
                             preferred_element_type=f32))
    h7 = jax.nn.relu(jnp.dot(h6 + jnp.dot(C2, h6) / deg6, W7,
                             preferred_element_type=f32))

    hp = jnp.take(h7, _final_perm(), axis=0).reshape(64, 64, 64)
    pooled3 = jnp.max(hp, axis=1)
    return _linear(pooled3.reshape(-1), Wlin, blin)


# speedup vs baseline: 11.3800x; 1.0642x over previous
import functools
import numpy as np
import jax
import jax.numpy as jnp
from jax import lax
from jax.experimental import pallas as pl
from jax.experimental.pallas import tpu as pltpu
from jax.experimental.pallas import tpu_sc as plsc

_NS = 50000
_SPAD = 50048
_DUMMY = 50047
_BLK = 128
_ZCH = 136


def _sc_segsum(table, srci, dsti, F, luts=None):
    E = srci.shape[0]
    ept = E // 32
    nb = ept // _BLK
    rows_pt = _SPAD // 16
    nz = rows_pt // _ZCH
    mesh = plsc.VectorSubcoreMesh(core_axis_name="c", subcore_axis_name="s")

    agg_ty = jax.ShapeDtypeStruct((2 * _SPAD, F), jnp.float32)
    e_i32 = jax.ShapeDtypeStruct((E,), jnp.int32)
    out_ty = agg_ty if luts is None else (agg_ty, e_i32, e_i32, e_i32)
    remap_scratch = [] if luts is None else [
        pltpu.VMEM((_BLK,), jnp.int32),
        pltpu.VMEM((_BLK,), jnp.int32),
        pltpu.VMEM((_BLK,), jnp.int32),
    ]

    @functools.partial(
        pl.kernel, mesh=mesh,
        compiler_params=pltpu.CompilerParams(use_tc_tiling_on_sc=False),
        out_type=out_ty,
        scratch_types=[
            pltpu.VMEM((_BLK,), jnp.int32),
            pltpu.VMEM((_BLK,), jnp.int32),
            pltpu.VMEM((_BLK, F), jnp.float32),
            pltpu.VMEM((_ZCH, F), jnp.float32),
            pltpu.VMEM_SHARED((_SPAD, F), jnp.float32),
            pltpu.SemaphoreType.DMA,
        ] + remap_scratch,
    )
    def k(t_hbm, src_hbm, dst_hbm, *rest):
        if luts is None:
            out_hbm, idxs_v, idxd_v, rows_v, zer_v, acc_sh, sem = rest
        else:
            (lut_hbm, out_hbm, cs_hbm, cd_hbm, fl_hbm,
             idxs_v, idxd_v, rows_v, zer_v, acc_sh, sem,
             ls_v, ld_v, fl_v) = rest
        cid = lax.axis_index("c")
        sid = lax.axis_index("s")

        def zfill(i, _):
            r = i // (F // 16)
            c = (i % (F // 16)) * 16
            zer_v[r, pl.ds(c, 16)] = jnp.zeros((16,), jnp.float32)
            return 0
        lax.fori_loop(0, _ZCH * (F // 16), zfill, 0)

        row0 = sid * rows_pt

        def zcopy(i, _):
            pltpu.sync_copy(zer_v, acc_sh.at[pl.ds(row0 + i * _ZCH, _ZCH)])
            return 0
        lax.fori_loop(0, nz, zcopy, 0)
        plsc.subcore_barrier()

        base = (cid * 16 + sid) * ept

        def body(b, _):
            off = base + b * _BLK
            pltpu.sync_copy(src_hbm.at[pl.ds(off, _BLK)], idxs_v)
            pltpu.sync_copy(dst_hbm.at[pl.ds(off, _BLK)], idxd_v)
            pltpu.async_copy(t_hbm.at[idxs_v], rows_v, sem).wait()
            pltpu.sync_copy(rows_v, acc_sh.at[idxd_v], add=True)
            if luts is not None:
                pltpu.async_copy(lut_hbm.at[idxs_v], ls_v, sem).wait()
                pltpu.async_copy(lut_hbm.at[idxd_v], ld_v, sem).wait()
                for j in range(_BLK // 16):
                    s16 = pl.ds(j * 16, 16)
                    ls, ld = ls_v[s16], ld_v[s16]
                    fl_v[s16] = (ld & 4095) * 4096 + (ls & 4095)
                    ls_v[s16] = ls >> 12
                    ld_v[s16] = ld >> 12
                pltpu.sync_copy(fl_v, fl_hbm.at[pl.ds(off, _BLK)])
                pltpu.sync_copy(ls_v, cs_hbm.at[pl.ds(off, _BLK)])
                pltpu.sync_copy(ld_v, cd_hbm.at[pl.ds(off, _BLK)])
            return 0
        lax.fori_loop(0, nb, body, 0)
        plsc.subcore_barrier()

        obase = cid * _SPAD + row0

        def wcopy(i, _):
            pltpu.sync_copy(acc_sh.at[pl.ds(row0 + i * _ZCH, _ZCH)],
                            out_hbm.at[pl.ds(obase + i * _ZCH, _ZCH)])
            return 0
        lax.fori_loop(0, nz, wcopy, 0)

    if luts is None:
        return k(table, srci, dsti)
    return k(table, srci, dsti, luts)


_NLUT = 53248


def _sc_make_lut(vid_pad, cidx_of_voxel):
    npt = _NLUT // 32
    mesh = plsc.VectorSubcoreMesh(core_axis_name="c", subcore_axis_name="s")

    @functools.partial(
        pl.kernel, mesh=mesh,
        compiler_params=pltpu.CompilerParams(use_tc_tiling_on_sc=False),
        out_type=jax.ShapeDtypeStruct((_NLUT,), jnp.int32),
        scratch_types=[
            pltpu.VMEM((_BLK,), jnp.int32),
            pltpu.VMEM((_BLK,), jnp.int32),
            pltpu.SemaphoreType.DMA,
        ],
    )
    def k(vid_hbm, cidx_hbm, out_hbm, vid_v, cs_v, sem):
        cid = lax.axis_index("c")
        sid = lax.axis_index("s")
        base = (cid * 16 + sid) * npt

        def body(b, _):
            off = base + b * _BLK
            pltpu.sync_copy(vid_hbm.at[pl.ds(off, _BLK)], vid_v)
            pltpu.async_copy(cidx_hbm.at[vid_v], cs_v, sem).wait()
            for j in range(_BLK // 16):
                s16 = pl.ds(j * 16, 16)
                v = vid_v[s16]
                vid2 = (((v >> 12) >> 2) * 256 + (((v >> 6) & 63) >> 2) * 16
                        + ((v & 63) >> 2))
                vid_v[s16] = (cs_v[s16] << 12) + vid2
            pltpu.sync_copy(vid_v, out_hbm.at[pl.ds(off, _BLK)])
            return 0
        lax.fori_loop(0, npt // _BLK, body, 0)

    return k(vid_pad, cidx_of_voxel)


def _segsum(table, srci, dsti, F, luts=None):
    r = _sc_segsum(table, srci, dsti, F, luts)
    p = (r if luts is None else r[0]).reshape(2, _SPAD, F)
    agg = (p[0] + p[1])[:_NS]
    return agg if luts is None else (agg, r[1], r[2], r[3])


def _pad_edges(s, d, e_pad):
    npad = e_pad - s.shape[0]
    return (jnp.concatenate([s, jnp.zeros((npad,), s.dtype)]),
            jnp.concatenate([d, jnp.full((npad,), _DUMMY, d.dtype)]))


def _seg_max0(vals, seg, n):
    m = jax.ops.segment_max(vals, seg, num_segments=n)
    return jnp.where(jnp.isfinite(m), m, 0.0)


def _linear_body(x_ref, w_ref, b_ref, o_ref):
    o_ref[...] = jnp.dot(x_ref[...], w_ref[...],
                         preferred_element_type=jnp.float32) + b_ref[...]


def _linear(flat, Wlin, blin):
    return pl.pallas_call(
        _linear_body,
        out_shape=jax.ShapeDtypeStruct((1, 100), jnp.float32),
    )(flat.reshape(1, -1), Wlin, blin.reshape(1, -1)).reshape(-1)


def _final_perm():
    r = np.arange(4096)
    x, y, z = r // 256, (r // 16) % 16, r % 16
    j = (x // 4) * 16 + (y // 4) * 4 + (z // 4)
    i = (x % 4) * 16 + (y % 4) * 4 + (z % 4)
    perm = np.zeros(4096, dtype=np.int32)
    perm[j * 64 + i] = r
    return jnp.asarray(perm)


def kernel(node, features, edges, W1, W2, W3, W4, W5, W6, W7, Wlin, blin):
    src, dst = edges[0], edges[1]
    nn = node.shape[0]
    f32 = jnp.float32

    c = node // 4
    vid = (c[:, 0] * 4096 + c[:, 1] * 64 + c[:, 2]).astype(jnp.int32)
    occ = jnp.zeros((262144,), jnp.int32).at[vid].set(1)
    cidx_of_voxel = jnp.cumsum(occ) - occ
    vid_pad = jnp.concatenate([vid, jnp.zeros((_NLUT - nn,), jnp.int32)])
    luts = _sc_make_lut(vid_pad, cidx_of_voxel)[:_SPAD]
    cid_lut = luts[:nn] >> 12
    vid2_lut = luts[:nn] & 4095

    srcA, dstA = _pad_edges(src, dst, 32 * _BLK * 200)
    zeros14 = jnp.zeros((nn, 14), f32)
    ones1 = jnp.ones((nn, 1), f32)
    tA = jnp.concatenate([features, ones1, zeros14], axis=1)
    sA, csrcA, cdstA, flatA = _segsum(tA, srcA, dstA, 16, luts=luts)
    deg_a = jnp.maximum(sA[:, 1:2], 1.0)
    z = features + sA[:, 0:1] / deg_a
    u = jnp.concatenate([jax.nn.relu(z), jax.nn.relu(-z), ones1, zeros14[:, :13]],
                        axis=1)
    sU = _segsum(u, srcA, dstA, 16)
    u2 = u[:, 0:2]
    m2 = jnp.concatenate([jnp.maximum(W1, 0.0), -jnp.minimum(W1, 0.0)], axis=0)
    h2 = jax.nn.relu(jnp.dot(u2 + sU[:, 0:2] / deg_a, jnp.dot(m2, W2),
                             preferred_element_type=f32))

    pooled1 = _seg_max0(h2, cid_lut, _NS)
    slot_vid2 = jnp.full((_NS,), 4096, jnp.int32).at[cid_lut].set(vid2_lut)

    loop = jnp.arange(_NS, dtype=src.dtype)
    bsrc = jnp.concatenate([csrcA[:src.shape[0]], loop])
    bdst = jnp.concatenate([cdstA[:src.shape[0]], loop])
    srcB, dstB = _pad_edges(bsrc, bdst, 32 * _BLK * 208)

    tB = jnp.concatenate([pooled1, jnp.ones((_NS, 1), f32),
                          jnp.zeros((_NS, 15), f32)], axis=1)
    s3 = _segsum(tB, srcB, dstB, 32)
    deg_b = jnp.maximum(s3[:, 16:17], 1.0)
    h3 = jax.nn.relu(jnp.dot(pooled1 + s3[:, :16] / deg_b, W3,
                             preferred_element_type=f32))
    s4 = _segsum(h3, srcB, dstB, 32)
    h4 = jax.nn.relu(jnp.dot(h3 + s4 / deg_b, W4, preferred_element_type=f32))
    s5 = _segsum(h4, srcB, dstB, 32)
    h5 = jax.nn.relu(jnp.dot(h4 + s5 / deg_b, W5, preferred_element_type=f32))

    pooled2 = _seg_max0(h5, slot_vid2, 4097)[:4096]

    flat_id = flatA[:src.shape[0]]
    C2 = jax.ops.segment_sum(jnp.ones_like(flat_id, dtype=f32), flat_id,
                             num_segments=4096 * 4096).reshape(4096, 4096)
    deg6 = jnp.maximum(C2.sum(axis=1, keepdims=True) + 64.0, 1.0)
    h6 = jax.nn.relu(jnp.dot(
        pooled2 + (jnp.dot(C2, pooled2) + 64.0 * pooled2) / deg6, W6,
        preferred_element_type=f32))
    h7 = jax.nn.relu(jnp.dot(h6 + (jnp.dot(C2, h6) + 64.0 * h6) / deg6, W7,
                             preferred_element_type=f32))

    hp = jnp.take(h7, _final_perm(), axis=0).reshape(64, 64, 64)
    pooled3 = jnp.max(hp, axis=1)
    return _linear(pooled3.reshape(-1), Wlin, blin)


# speedup vs baseline: 14.3444x; 1.2605x over previous
import functools
import numpy as np
import jax
import jax.numpy as jnp
from jax import lax
from jax.experimental import pallas as pl
from jax.experimental.pallas import tpu as pltpu
from jax.experimental.pallas import tpu_sc as plsc

_NS = 50000
_SPAD = 50048
_DUMMY = 50047
_BLK = 128
_ZCH = 136


def _sc_segsum(table, srci, dsti, F, luts=None):
    E = srci.shape[0]
    ept = E // 32
    nb = ept // _BLK
    rows_pt = _SPAD // 16
    nz = rows_pt // _ZCH
    mesh = plsc.VectorSubcoreMesh(core_axis_name="c", subcore_axis_name="s")

    agg_ty = jax.ShapeDtypeStruct((2 * _SPAD, F), jnp.float32)
    e_i32 = jax.ShapeDtypeStruct((E,), jnp.int32)
    out_ty = agg_ty if luts is None else (agg_ty, e_i32, e_i32, e_i32)
    remap_scratch = [] if luts is None else [
        pltpu.VMEM((_BLK,), jnp.int32),
        pltpu.VMEM((_BLK,), jnp.int32),
        pltpu.VMEM((_BLK,), jnp.int32),
    ]

    @functools.partial(
        pl.kernel, mesh=mesh,
        compiler_params=pltpu.CompilerParams(use_tc_tiling_on_sc=False),
        out_type=out_ty,
        scratch_types=[
            pltpu.VMEM((2, _BLK), jnp.int32),
            pltpu.VMEM((2, _BLK), jnp.int32),
            pltpu.VMEM((2, _BLK, F), jnp.float32),
            pltpu.VMEM((_ZCH, F), jnp.float32),
            pltpu.VMEM_SHARED((_SPAD, F), jnp.float32),
            pltpu.SemaphoreType.DMA,
            pltpu.SemaphoreType.DMA,
            pltpu.SemaphoreType.DMA,
            pltpu.SemaphoreType.DMA,
        ] + remap_scratch,
    )
    def k(t_hbm, src_hbm, dst_hbm, *rest):
        if luts is None:
            (out_hbm, idxs_v, idxd_v, rows_v, zer_v, acc_sh,
             semi0, semi1, semg0, semg1) = rest
        else:
            (lut_hbm, out_hbm, cs_hbm, cd_hbm, fl_hbm,
             idxs_v, idxd_v, rows_v, zer_v, acc_sh,
             semi0, semi1, semg0, semg1,
             ls_v, ld_v, fl_v) = rest
        semi = (semi0, semi1)
        semg = (semg0, semg1)
        cid = lax.axis_index("c")
        sid = lax.axis_index("s")

        def zfill(i, _):
            r = i // (F // 16)
            c = (i % (F // 16)) * 16
            zer_v[r, pl.ds(c, 16)] = jnp.zeros((16,), jnp.float32)
            return 0
        lax.fori_loop(0, _ZCH * (F // 16), zfill, 0)

        row0 = sid * rows_pt

        def zcopy(i, _):
            pltpu.sync_copy(zer_v, acc_sh.at[pl.ds(row0 + i * _ZCH, _ZCH)])
            return 0
        lax.fori_loop(0, nz, zcopy, 0)
        plsc.subcore_barrier()

        base = (cid * 16 + sid) * ept

        def _idx_copies(b, p):
            off = base + b * _BLK
            return (pltpu.make_async_copy(src_hbm.at[pl.ds(off, _BLK)],
                                          idxs_v.at[p], semi[p]),
                    pltpu.make_async_copy(dst_hbm.at[pl.ds(off, _BLK)],
                                          idxd_v.at[p], semi[p]))

        for p in range(2):
            for cp in _idx_copies(p, p):
                cp.start()

        def body(i, _):
            b0 = i * 2
            for p in range(2):
                for cp in _idx_copies(b0 + p, p):
                    cp.wait()
                pltpu.async_copy(t_hbm.at[idxs_v.at[p]], rows_v.at[p], semg[p])
            for p in range(2):
                b = b0 + p
                off = base + b * _BLK
                pltpu.make_async_copy(t_hbm.at[idxs_v.at[p]], rows_v.at[p],
                                      semg[p]).wait()
                if luts is not None:
                    gls = pltpu.make_async_copy(lut_hbm.at[idxs_v.at[p]], ls_v,
                                                semg[p])
                    gld = pltpu.make_async_copy(lut_hbm.at[idxd_v.at[p]], ld_v,
                                                semg[p])
                    gls.start()
                    gld.start()
                    gls.wait()
                    gld.wait()
                    for j in range(_BLK // 16):
                        s16 = pl.ds(j * 16, 16)
                        ls, ld = ls_v[s16], ld_v[s16]
                        fl_v[s16] = (ld & 4095) * 4096 + (ls & 4095)
                        ls_v[s16] = ls >> 12
                        ld_v[s16] = ld >> 12
                    pltpu.sync_copy(fl_v, fl_hbm.at[pl.ds(off, _BLK)])
                    pltpu.sync_copy(ls_v, cs_hbm.at[pl.ds(off, _BLK)])
                    pltpu.sync_copy(ld_v, cd_hbm.at[pl.ds(off, _BLK)])
                pltpu.sync_copy(rows_v.at[p], acc_sh.at[idxd_v.at[p]], add=True)

                @pl.when(b + 2 < nb)
                def _():
                    for cp in _idx_copies(b + 2, p):
                        cp.start()
            return 0
        lax.fori_loop(0, nb // 2, body, 0)
        plsc.subcore_barrier()

        obase = cid * _SPAD + row0

        def wcopy(i, _):
            pltpu.sync_copy(acc_sh.at[pl.ds(row0 + i * _ZCH, _ZCH)],
                            out_hbm.at[pl.ds(obase + i * _ZCH, _ZCH)])
            return 0
        lax.fori_loop(0, nz, wcopy, 0)

    if luts is None:
        return k(table, srci, dsti)
    return k(table, srci, dsti, luts)


_NLUT = 53248


def _sc_make_lut(vid_pad, cidx_of_voxel):
    npt = _NLUT // 32
    mesh = plsc.VectorSubcoreMesh(core_axis_name="c", subcore_axis_name="s")

    @functools.partial(
        pl.kernel, mesh=mesh,
        compiler_params=pltpu.CompilerParams(use_tc_tiling_on_sc=False),
        out_type=jax.ShapeDtypeStruct((_NLUT,), jnp.int32),
        scratch_types=[
            pltpu.VMEM((_BLK,), jnp.int32),
            pltpu.VMEM((_BLK,), jnp.int32),
            pltpu.SemaphoreType.DMA,
        ],
    )
    def k(vid_hbm, cidx_hbm, out_hbm, vid_v, cs_v, sem):
        cid = lax.axis_index("c")
        sid = lax.axis_index("s")
        base = (cid * 16 + sid) * npt

        def body(b, _):
            off = base + b * _BLK
            pltpu.sync_copy(vid_hbm.at[pl.ds(off, _BLK)], vid_v)
            pltpu.async_copy(cidx_hbm.at[vid_v], cs_v, sem).wait()
            for j in range(_BLK // 16):
                s16 = pl.ds(j * 16, 16)
                v = vid_v[s16]
                vid2 = (((v >> 12) >> 2) * 256 + (((v >> 6) & 63) >> 2) * 16
                        + ((v & 63) >> 2))
                vid_v[s16] = (cs_v[s16] << 12) + vid2
            pltpu.sync_copy(vid_v, out_hbm.at[pl.ds(off, _BLK)])
            return 0
        lax.fori_loop(0, npt // _BLK, body, 0)

    return k(vid_pad, cidx_of_voxel)


def _segsum(table, srci, dsti, F, luts=None):
    r = _sc_segsum(table, srci, dsti, F, luts)
    p = (r if luts is None else r[0]).reshape(2, _SPAD, F)
    agg = (p[0] + p[1])[:_NS]
    return agg if luts is None else (agg, r[1], r[2], r[3])


def _pad_edges(s, d, e_pad):
    npad = e_pad - s.shape[0]
    return (jnp.concatenate([s, jnp.zeros((npad,), s.dtype)]),
            jnp.concatenate([d, jnp.full((npad,), _DUMMY, d.dtype)]))


def _seg_max0(vals, seg, n):
    m = jax.ops.segment_max(vals, seg, num_segments=n)
    return jnp.where(jnp.isfinite(m), m, 0.0)


def _linear_body(x_ref, w_ref, b_ref, o_ref):
    o_ref[...] = jnp.dot(x_ref[...], w_ref[...],
                         preferred_element_type=jnp.float32) + b_ref[...]


def _linear(flat, Wlin, blin):
    return pl.pallas_call(
        _linear_body,
        out_shape=jax.ShapeDtypeStruct((1, 100), jnp.float32),
    )(flat.reshape(1, -1), Wlin, blin.reshape(1, -1)).reshape(-1)


def _final_perm():
    r = np.arange(4096)
    x, y, z = r // 256, (r // 16) % 16, r % 16
    j = (x // 4) * 16 + (y // 4) * 4 + (z // 4)
    i = (x % 4) * 16 + (y % 4) * 4 + (z % 4)
    perm = np.zeros(4096, dtype=np.int32)
    perm[j * 64 + i] = r
    return jnp.asarray(perm)


def kernel(node, features, edges, W1, W2, W3, W4, W5, W6, W7, Wlin, blin):
    src, dst = edges[0], edges[1]
    nn = node.shape[0]
    f32 = jnp.float32

    c = node // 4
    vid = (c[:, 0] * 4096 + c[:, 1] * 64 + c[:, 2]).astype(jnp.int32)
    occ = jnp.zeros((262144,), jnp.int32).at[vid].set(1)
    cidx_of_voxel = jnp.cumsum(occ) - occ
    vid_pad = jnp.concatenate([vid, jnp.zeros((_NLUT - nn,), jnp.int32)])
    luts = _sc_make_lut(vid_pad, cidx_of_voxel)[:_SPAD]
    cid_lut = luts[:nn] >> 12
    vid2_lut = luts[:nn] & 4095

    srcA, dstA = _pad_edges(src, dst, 32 * _BLK * 200)
    zeros14 = jnp.zeros((nn, 14), f32)
    ones1 = jnp.ones((nn, 1), f32)
    tA = jnp.concatenate([features, ones1, zeros14], axis=1)
    sA, csrcA, cdstA, flatA = _segsum(tA, srcA, dstA, 16, luts=luts)
    deg_a = jnp.maximum(sA[:, 1:2], 1.0)
    z = features + sA[:, 0:1] / deg_a
    u = jnp.concatenate([jax.nn.relu(z), jax.nn.relu(-z), ones1, zeros14[:, :13]],
                        axis=1)
    sU = _segsum(u, srcA, dstA, 16)
    u2 = u[:, 0:2]
    m2 = jnp.concatenate([jnp.maximum(W1, 0.0), -jnp.minimum(W1, 0.0)], axis=0)
    h2 = jax.nn.relu(jnp.dot(u2 + sU[:, 0:2] / deg_a, jnp.dot(m2, W2),
                             preferred_element_type=f32))

    pooled1 = _seg_max0(h2, cid_lut, _NS)
    slot_vid2 = jnp.full((_NS,), 4096, jnp.int32).at[cid_lut].set(vid2_lut)

    loop = jnp.arange(_NS, dtype=src.dtype)
    bsrc = jnp.concatenate([csrcA[:src.shape[0]], loop])
    bdst = jnp.concatenate([cdstA[:src.shape[0]], loop])
    srcB, dstB = _pad_edges(bsrc, bdst, 32 * _BLK * 208)

    tB = jnp.concatenate([pooled1, jnp.ones((_NS, 1), f32),
                          jnp.zeros((_NS, 15), f32)], axis=1)
    s3 = _segsum(tB, srcB, dstB, 32)
    deg_b = jnp.maximum(s3[:, 16:17], 1.0)
    h3 = jax.nn.relu(jnp.dot(pooled1 + s3[:, :16] / deg_b, W3,
                             preferred_element_type=f32))
    s4 = _segsum(h3, srcB, dstB, 32)
    h4 = jax.nn.relu(jnp.dot(h3 + s4 / deg_b, W4, preferred_element_type=f32))
    s5 = _segsum(h4, srcB, dstB, 32)
    h5 = jax.nn.relu(jnp.dot(h4 + s5 / deg_b, W5, preferred_element_type=f32))

    pooled2 = _seg_max0(h5, slot_vid2, 4097)[:4096]

    flat_id = flatA[:src.shape[0]]
    C2 = jax.ops.segment_sum(jnp.ones_like(flat_id, dtype=f32), flat_id,
                             num_segments=4096 * 4096).reshape(4096, 4096)
    deg6 = jnp.maximum(C2.sum(axis=1, keepdims=True) + 64.0, 1.0)
    h6 = jax.nn.relu(jnp.dot(
        pooled2 + (jnp.dot(C2, pooled2) + 64.0 * pooled2) / deg6, W6,
        preferred_element_type=f32))
    h7 = jax.nn.relu(jnp.dot(h6 + (jnp.dot(C2, h6) + 64.0 * h6) / deg6, W7,
                             preferred_element_type=f32))

    hp = jnp.take(h7, _final_perm(), axis=0).reshape(64, 64, 64)
    pooled3 = jnp.max(hp, axis=1)
    return _linear(pooled3.reshape(-1), Wlin, blin)


# speedup vs baseline: 15.9279x; 1.1104x over previous
import functools
import numpy as np
import jax
import jax.numpy as jnp
from jax import lax
from jax.experimental import pallas as pl
from jax.experimental.pallas import tpu as pltpu
from jax.experimental.pallas import tpu_sc as plsc

_NS = 50000
_SPAD = 50048
_DUMMY = 50047
_BLK = 128
_EA = 32 * _BLK * 200
_ZCH = 136


def _sc_segsum(table, srci, dsti, F, luts=None, spad=_SPAD, zch=_ZCH):
    E = srci.shape[0]
    ept = E // 32
    nb = ept // _BLK
    rows_pt = spad // 16
    nz = rows_pt // zch
    mesh = plsc.VectorSubcoreMesh(core_axis_name="c", subcore_axis_name="s")

    agg_ty = jax.ShapeDtypeStruct((2 * spad, F), jnp.float32)
    e_i32 = jax.ShapeDtypeStruct((E,), jnp.int32)
    out_ty = agg_ty if luts is None else (agg_ty, e_i32, e_i32, e_i32)
    remap_scratch = [] if luts is None else [
        pltpu.VMEM((_BLK,), jnp.int32),
        pltpu.VMEM((_BLK,), jnp.int32),
        pltpu.VMEM((_BLK,), jnp.int32),
    ]

    @functools.partial(
        pl.kernel, mesh=mesh,
        compiler_params=pltpu.CompilerParams(use_tc_tiling_on_sc=False),
        out_type=out_ty,
        scratch_types=[
            pltpu.VMEM((2, _BLK), jnp.int32),
            pltpu.VMEM((2, _BLK), jnp.int32),
            pltpu.VMEM((2, _BLK, F), jnp.float32),
            pltpu.VMEM((zch, F), jnp.float32),
            pltpu.VMEM_SHARED((spad, F), jnp.float32),
            pltpu.SemaphoreType.DMA,
            pltpu.SemaphoreType.DMA,
            pltpu.SemaphoreType.DMA,
            pltpu.SemaphoreType.DMA,
        ] + remap_scratch,
    )
    def k(t_hbm, src_hbm, dst_hbm, *rest):
        if luts is None:
            (out_hbm, idxs_v, idxd_v, rows_v, zer_v, acc_sh,
             semi0, semi1, semg0, semg1) = rest
        else:
            (lut_hbm, out_hbm, cs_hbm, cd_hbm, fl_hbm,
             idxs_v, idxd_v, rows_v, zer_v, acc_sh,
             semi0, semi1, semg0, semg1,
             ls_v, ld_v, fl_v) = rest
        semi = (semi0, semi1)
        semg = (semg0, semg1)
        cid = lax.axis_index("c")
        sid = lax.axis_index("s")

        def zfill(i, _):
            r = i // (F // 16)
            c = (i % (F // 16)) * 16
            zer_v[r, pl.ds(c, 16)] = jnp.zeros((16,), jnp.float32)
            return 0
        lax.fori_loop(0, zch * (F // 16), zfill, 0)

        row0 = sid * rows_pt

        def zcopy(i, _):
            pltpu.sync_copy(zer_v, acc_sh.at[pl.ds(row0 + i * zch, zch)])
            return 0
        lax.fori_loop(0, nz, zcopy, 0)
        plsc.subcore_barrier()

        base = (cid * 16 + sid) * ept

        def _idx_copies(b, p):
            off = base + b * _BLK
            return (pltpu.make_async_copy(src_hbm.at[pl.ds(off, _BLK)],
                                          idxs_v.at[p], semi[p]),
                    pltpu.make_async_copy(dst_hbm.at[pl.ds(off, _BLK)],
                                          idxd_v.at[p], semi[p]))

        for p in range(2):
            for cp in _idx_copies(p, p):
                cp.start()

        def body(i, _):
            b0 = i * 2
            for p in range(2):
                for cp in _idx_copies(b0 + p, p):
                    cp.wait()
                pltpu.async_copy(t_hbm.at[idxs_v.at[p]], rows_v.at[p], semg[p])
            for p in range(2):
                b = b0 + p
                off = base + b * _BLK
                pltpu.make_async_copy(t_hbm.at[idxs_v.at[p]], rows_v.at[p],
                                      semg[p]).wait()
                if luts is not None:
                    gls = pltpu.make_async_copy(lut_hbm.at[idxs_v.at[p]], ls_v,
                                                semg[p])
                    gld = pltpu.make_async_copy(lut_hbm.at[idxd_v.at[p]], ld_v,
                                                semg[p])
                    gls.start()
                    gld.start()
                    gls.wait()
                    gld.wait()
                    for j in range(_BLK // 16):
                        s16 = pl.ds(j * 16, 16)
                        ls, ld = ls_v[s16], ld_v[s16]
                        fl_v[s16] = (ld & 4095) * 4096 + (ls & 4095)
                        ls_v[s16] = ls >> 12
                        ld_v[s16] = ld >> 12
                    pltpu.sync_copy(fl_v, fl_hbm.at[pl.ds(off, _BLK)])
                    pltpu.sync_copy(ls_v, cs_hbm.at[pl.ds(off, _BLK)])
                    pltpu.sync_copy(ld_v, cd_hbm.at[pl.ds(off, _BLK)])
                pltpu.sync_copy(rows_v.at[p], acc_sh.at[idxd_v.at[p]], add=True)

                @pl.when(b + 2 < nb)
                def _():
                    for cp in _idx_copies(b + 2, p):
                        cp.start()
            return 0
        lax.fori_loop(0, nb // 2, body, 0)
        plsc.subcore_barrier()

        obase = cid * spad + row0

        def wcopy(i, _):
            pltpu.sync_copy(acc_sh.at[pl.ds(row0 + i * zch, zch)],
                            out_hbm.at[pl.ds(obase + i * zch, zch)])
            return 0
        lax.fori_loop(0, nz, wcopy, 0)

    if luts is None:
        return k(table, srci, dsti)
    return k(table, srci, dsti, luts)


_NLUT = 53248


def _sc_make_lut(vid_pad, cidx_of_voxel):
    npt = _NLUT // 32
    mesh = plsc.VectorSubcoreMesh(core_axis_name="c", subcore_axis_name="s")

    @functools.partial(
        pl.kernel, mesh=mesh,
        compiler_params=pltpu.CompilerParams(use_tc_tiling_on_sc=False),
        out_type=jax.ShapeDtypeStruct((_NLUT,), jnp.int32),
        scratch_types=[
            pltpu.VMEM((_BLK,), jnp.int32),
            pltpu.VMEM((_BLK,), jnp.int32),
            pltpu.SemaphoreType.DMA,
        ],
    )
    def k(vid_hbm, cidx_hbm, out_hbm, vid_v, cs_v, sem):
        cid = lax.axis_index("c")
        sid = lax.axis_index("s")
        base = (cid * 16 + sid) * npt

        def body(b, _):
            off = base + b * _BLK
            pltpu.sync_copy(vid_hbm.at[pl.ds(off, _BLK)], vid_v)
            pltpu.async_copy(cidx_hbm.at[vid_v], cs_v, sem).wait()
            for j in range(_BLK // 16):
                s16 = pl.ds(j * 16, 16)
                v = vid_v[s16]
                vid2 = (((v >> 12) >> 2) * 256 + (((v >> 6) & 63) >> 2) * 16
                        + ((v & 63) >> 2))
                vid_v[s16] = (cs_v[s16] << 12) + vid2
            pltpu.sync_copy(vid_v, out_hbm.at[pl.ds(off, _BLK)])
            return 0
        lax.fori_loop(0, npt // _BLK, body, 0)

    return k(vid_pad, cidx_of_voxel)


def _segsum(table, srci, dsti, F, luts=None, spad=_SPAD, zch=_ZCH, ns=_NS):
    r = _sc_segsum(table, srci, dsti, F, luts, spad=spad, zch=zch)
    p = (r if luts is None else r[0]).reshape(2, spad, F)
    agg = (p[0] + p[1])[:ns]
    return agg if luts is None else (agg, r[1], r[2], r[3])


def _pad_edges(s, d, e_pad):
    npad = e_pad - s.shape[0]
    return (jnp.concatenate([s, jnp.zeros((npad,), s.dtype)]),
            jnp.concatenate([d, jnp.full((npad,), _DUMMY, d.dtype)]))


def _seg_max0(vals, seg, n):
    m = jax.ops.segment_max(vals, seg, num_segments=n)
    return jnp.where(jnp.isfinite(m), m, 0.0)


def _linear_body(x_ref, w_ref, b_ref, o_ref):
    o_ref[...] = jnp.dot(x_ref[...], w_ref[...],
                         preferred_element_type=jnp.float32) + b_ref[...]


def _linear(flat, Wlin, blin):
    return pl.pallas_call(
        _linear_body,
        out_shape=jax.ShapeDtypeStruct((1, 100), jnp.float32),
    )(flat.reshape(1, -1), Wlin, blin.reshape(1, -1)).reshape(-1)


def _final_perm():
    r = np.arange(4096)
    x, y, z = r // 256, (r // 16) % 16, r % 16
    j = (x // 4) * 16 + (y // 4) * 4 + (z // 4)
    i = (x % 4) * 16 + (y % 4) * 4 + (z % 4)
    perm = np.zeros(4096, dtype=np.int32)
    perm[j * 64 + i] = r
    return jnp.asarray(perm)


def kernel(node, features, edges, W1, W2, W3, W4, W5, W6, W7, Wlin, blin):
    src, dst = edges[0], edges[1]
    nn = node.shape[0]
    f32 = jnp.float32

    c = node // 4
    vid = (c[:, 0] * 4096 + c[:, 1] * 64 + c[:, 2]).astype(jnp.int32)
    occ = jnp.zeros((262144,), jnp.int32).at[vid].set(1)
    cidx_of_voxel = jnp.cumsum(occ) - occ
    vid_pad = jnp.concatenate([vid, jnp.zeros((_NLUT - nn,), jnp.int32)])
    luts = _sc_make_lut(vid_pad, cidx_of_voxel)[:_SPAD]
    cid_lut = luts[:nn] >> 12
    vid2_lut = luts[:nn] & 4095

    srcA, dstA = _pad_edges(src, dst, _EA)
    zeros14 = jnp.zeros((nn, 14), f32)
    ones1 = jnp.ones((nn, 1), f32)
    tA = jnp.concatenate([features, ones1, zeros14], axis=1)
    sA, csrcA, cdstA, flatA = _segsum(tA, srcA, dstA, 16, luts=luts)
    deg_a = jnp.maximum(sA[:, 1:2], 1.0)
    z = features + sA[:, 0:1] / deg_a
    u = jnp.concatenate([jax.nn.relu(z), jax.nn.relu(-z), ones1, zeros14[:, :13]],
                        axis=1)
    sU = _segsum(u, srcA, dstA, 16)
    u2 = u[:, 0:2]
    m2 = jnp.concatenate([jnp.maximum(W1, 0.0), -jnp.minimum(W1, 0.0)], axis=0)
    h2 = jax.nn.relu(jnp.dot(u2 + sU[:, 0:2] / deg_a, jnp.dot(m2, W2),
                             preferred_element_type=f32))

    pooled1 = _seg_max0(h2, cid_lut, _NS)
    slot_vid2 = jnp.full((_NS,), 4096, jnp.int32).at[cid_lut].set(vid2_lut)

    loop = jnp.arange(_NS, dtype=src.dtype)
    bsrc = jnp.concatenate([csrcA[:src.shape[0]], loop])
    bdst = jnp.concatenate([cdstA[:src.shape[0]], loop])
    srcB, dstB = _pad_edges(bsrc, bdst, 32 * _BLK * 208)

    tB = jnp.concatenate([pooled1, jnp.ones((_NS, 1), f32),
                          jnp.zeros((_NS, 15), f32)], axis=1)
    s3 = _segsum(tB, srcB, dstB, 32)
    deg_b = jnp.maximum(s3[:, 16:17], 1.0)
    h3 = jax.nn.relu(jnp.dot(pooled1 + s3[:, :16] / deg_b, W3,
                             preferred_element_type=f32))
    s4 = _segsum(h3, srcB, dstB, 32)
    h4 = jax.nn.relu(jnp.dot(h3 + s4 / deg_b, W4, preferred_element_type=f32))
    s5 = _segsum(h4, srcB, dstB, 32)
    h5 = jax.nn.relu(jnp.dot(h4 + s5 / deg_b, W5, preferred_element_type=f32))

    pooled2 = _seg_max0(h5, slot_vid2, 4097)[:4096]

    ne = src.shape[0]
    flat_id = flatA[:ne]
    sC, zC = 4160, 260
    dummy_c = 4159
    vs2 = jnp.concatenate([flat_id & 4095, jnp.zeros((_EA - ne,), jnp.int32)])
    vd2 = jnp.concatenate([flat_id >> 12,
                           jnp.full((_EA - ne,), dummy_c, jnp.int32)])

    def conv_c(h, W):
        aggL = _segsum(h[:, :32], vs2, vd2, 32, spad=sC, zch=zC, ns=4096)
        aggR = _segsum(h[:, 32:], vs2, vd2, 32, spad=sC, zch=zC, ns=4096)
        agg = jnp.concatenate([aggL, aggR], axis=1)
        return jax.nn.relu(jnp.dot(h + (agg + 64.0 * h) / deg6, W,
                                   preferred_element_type=f32))

    tD = jnp.concatenate([jnp.ones((4096, 1), f32), jnp.zeros((4096, 15), f32)],
                         axis=1)
    deg6 = jnp.maximum(
        _segsum(tD, vs2, vd2, 16, spad=sC, zch=zC, ns=4096)[:, 0:1] + 64.0, 1.0)
    h6 = conv_c(pooled2, W6)
    h7 = conv_c(h6, W7)

    hp = jnp.take(h7, _final_perm(), axis=0).reshape(64, 64, 64)
    pooled3 = jnp.max(hp, axis=1)
    return _linear(pooled3.reshape(-1), Wlin, blin)
